# Initial kernel scaffold; baseline (speedup 1.0000x reference)
import jax
import jax.numpy as jnp
from jax.experimental import pallas as pl


def kernel(xyz, embeddings, min_xyz, max_xyz):
    raise NotImplementedError("write your pallas kernel here")


# speedup vs baseline: 1.8666x; 1.8666x over previous
import dataclasses
import functools

import jax
import jax.numpy as jnp
from jax import lax
from jax.experimental import pallas as pl
from jax.experimental.pallas import tpu as pltpu
from jax.experimental.pallas import tpu_sc as plsc

N_LEVELS = 16
N_FEAT = 2
_OFFSETS = [0, 4913, 17080, 46871, 126378, 331757, 856045, 1380333, 1904621,
            2428909, 2953197, 3477485, 4001773, 4526061, 5050349, 5574637,
            6098925]
_RESOLUTIONS = [16, 22, 30, 42, 58, 80, 111, 154, 212, 294, 406, 561, 776,
                1072, 1482, 2048]
_PRIME1 = 2654435761
_PRIME2 = 805459861

_N_POINTS = 131072
_NC = 2
_NS = 16
_NW = _NC * _NS
_LANES = 16
_CHUNK = 128
_PTS_PER_W = _N_POINTS // _NW
_CHUNKS_PER_W = _PTS_PER_W // _CHUNK
_VPC = _CHUNK // _LANES
_LC = N_LEVELS * 8


def _level_meta():
    meta = []
    for l in range(N_LEVELS):
        res = _RESOLUTIONS[l]
        size = _OFFSETS[l + 1] - _OFFSETS[l]
        dense = (res + 1) ** 3 <= size
        meta.append((res, size, dense, _OFFSETS[l]))
    return meta

_META = _level_meta()


def _sc_body(xt_hbm, valid_hbm, emb0_hbm, emb1_hbm, out_hbm,
             xbuf, vbuf, fracbuf, idxbuf, r0buf, r1buf, outbuf, sem):
    wid = lax.axis_index("c") * _NS + lax.axis_index("s")
    iota = lax.iota(jnp.int32, _LANES)

    @pl.loop(0, _CHUNKS_PER_W)
    def _chunk(ci):
        pbase = wid * _PTS_PER_W + ci * _CHUNK
        for d in range(3):
            pltpu.sync_copy(xt_hbm.at[pl.ds(d * _N_POINTS + pbase, _CHUNK)],
                            xbuf.at[d])
        pltpu.sync_copy(valid_hbm.at[pl.ds(pbase, _CHUNK)], vbuf)

        for l in range(N_LEVELS):
            res, size, dense, off = _META[l]
            resf = jnp.float32(res)

            @pl.loop(0, _VPC)
            def _idx(v):
                sl = pl.ds(v * _LANES, _LANES)
                x = xbuf[0, sl]
                y = xbuf[1, sl]
                z = xbuf[2, sl]
                px = x * resf
                py = y * resf
                pz = z * resf
                ix = jnp.minimum(px.astype(jnp.int32), res - 1)
                iy = jnp.minimum(py.astype(jnp.int32), res - 1)
                iz = jnp.minimum(pz.astype(jnp.int32), res - 1)
                fracbuf[3 * l + 0, sl] = jnp.minimum(px - ix.astype(jnp.float32), 1.0)
                fracbuf[3 * l + 1, sl] = jnp.minimum(py - iy.astype(jnp.float32), 1.0)
                fracbuf[3 * l + 2, sl] = jnp.minimum(pz - iz.astype(jnp.float32), 1.0)
                if dense:
                    r1 = res + 1
                    base = (ix * r1 + iy) * r1 + iz + off
                    for corner in range(8):
                        cadd = ((corner & 1) * r1 * r1 + ((corner >> 1) & 1) * r1
                                + ((corner >> 2) & 1))
                        idxbuf[8 * l + corner, sl] = base + cadd
                else:
                    mask = jnp.uint32(size - 1)
                    hx0 = ix.astype(jnp.uint32)
                    hx1 = hx0 + jnp.uint32(1)
                    hy0 = iy.astype(jnp.uint32) * jnp.uint32(_PRIME1)
                    hy1 = hy0 + jnp.uint32(_PRIME1)
                    hz0 = iz.astype(jnp.uint32) * jnp.uint32(_PRIME2)
                    hz1 = hz0 + jnp.uint32(_PRIME2)
                    hxs = (hx0, hx1)
                    hys = (hy0, hy1)
                    hzs = (hz0, hz1)
                    for corner in range(8):
                        h = hxs[corner & 1] ^ hys[(corner >> 1) & 1] ^ hzs[(corner >> 2) & 1]
                        idxbuf[8 * l + corner, sl] = (h & mask).astype(jnp.int32) + off

        @pl.loop(0, _LC)
        def _fire(j):
            pltpu.async_copy(emb0_hbm.at[idxbuf.at[j]], r0buf.at[j], sem)
            pltpu.async_copy(emb1_hbm.at[idxbuf.at[j]], r1buf.at[j], sem)

        @pl.loop(0, _LC)
        def _drain(j):
            pltpu.make_async_copy(emb0_hbm.at[idxbuf.at[j]], r0buf.at[j],
                                  sem).wait()
            pltpu.make_async_copy(emb1_hbm.at[idxbuf.at[j]], r1buf.at[j],
                                  sem).wait()

        for l in range(N_LEVELS):
            @pl.loop(0, _VPC)
            def _interp(v):
                sl = pl.ds(v * _LANES, _LANES)
                fx = fracbuf[3 * l + 0, sl]
                fy = fracbuf[3 * l + 1, sl]
                fz = fracbuf[3 * l + 2, sl]
                vld = vbuf[sl]
                wx = (1.0 - fx, fx)
                wy = (1.0 - fy, fy)
                wz = ((1.0 - fz) * vld, fz * vld)
                f0 = jnp.zeros((_LANES,), jnp.float32)
                f1 = jnp.zeros((_LANES,), jnp.float32)
                for corner in range(8):
                    w = (wx[corner & 1] * wy[(corner >> 1) & 1]
                         * wz[(corner >> 2) & 1])
                    f0 = f0 + w * r0buf[8 * l + corner, sl]
                    f1 = f1 + w * r1buf[8 * l + corner, sl]
                pidx = iota + v * _LANES
                plsc.store_scatter(
                    outbuf, [pidx, jnp.full((_LANES,), 2 * l, jnp.int32)], f0)
                plsc.store_scatter(
                    outbuf, [pidx, jnp.full((_LANES,), 2 * l + 1, jnp.int32)], f1)

        pltpu.sync_copy(outbuf, out_hbm.at[pl.ds(pbase, _CHUNK)])


@jax.jit
def kernel(xyz, embeddings, min_xyz, max_xyz):
    scale = 1.0 / (max_xyz - min_xyz)
    xn = (xyz - min_xyz[None, :]) * scale[None, :]
    valid = jnp.all((xn >= 0.0) & (xn <= 1.0), axis=-1).astype(jnp.float32)
    xt = jnp.clip(xn, 0.0, 1.0).T.reshape(-1)
    emb0 = embeddings[:, 0]
    emb1 = embeddings[:, 1]

    mesh = plsc.VectorSubcoreMesh(core_axis_name="c", subcore_axis_name="s",
                                  num_cores=_NC, num_subcores=_NS)
    cp = pltpu.CompilerParams()
    if "needs_layout_passes" in pltpu.CompilerParams.__dataclass_fields__:
        cp = dataclasses.replace(cp, needs_layout_passes=False)
    if "use_tc_tiling_on_sc" in pltpu.CompilerParams.__dataclass_fields__:
        cp = dataclasses.replace(cp, use_tc_tiling_on_sc=False)
    run = pl.kernel(
        _sc_body,
        out_type=jax.ShapeDtypeStruct((_N_POINTS, 2 * N_LEVELS), jnp.float32),
        mesh=mesh,
        scratch_types=[
            pltpu.VMEM((3, _CHUNK), jnp.float32),
            pltpu.VMEM((_CHUNK,), jnp.float32),
            pltpu.VMEM((3 * N_LEVELS, _CHUNK), jnp.float32),
            pltpu.VMEM((_LC, _CHUNK), jnp.int32),
            pltpu.VMEM((_LC, _CHUNK), jnp.float32),
            pltpu.VMEM((_LC, _CHUNK), jnp.float32),
            pltpu.VMEM((_CHUNK, 2 * N_LEVELS), jnp.float32),
            pltpu.SemaphoreType.DMA,
        ],
        compiler_params=cp,
    )
    return run(xt, valid, emb0, emb1)


# speedup vs baseline: 2.7918x; 1.4957x over previous
import dataclasses
import functools

import jax
import jax.numpy as jnp
from jax import lax
from jax.experimental import pallas as pl
from jax.experimental.pallas import tpu as pltpu
from jax.experimental.pallas import tpu_sc as plsc

N_LEVELS = 16
N_FEAT = 2
_OFFSETS = [0, 4913, 17080, 46871, 126378, 331757, 856045, 1380333, 1904621,
            2428909, 2953197, 3477485, 4001773, 4526061, 5050349, 5574637,
            6098925]
_RESOLUTIONS = [16, 22, 30, 42, 58, 80, 111, 154, 212, 294, 406, 561, 776,
                1072, 1482, 2048]
_PRIME1 = 2654435761
_PRIME2 = 805459861

_N_POINTS = 131072
_NC = 2
_NS = 16
_NW = _NC * _NS
_LANES = 16
_CHUNK = 128
_PTS_PER_W = _N_POINTS // _NW
_CHUNKS_PER_W = _PTS_PER_W // _CHUNK
_VPC = _CHUNK // _LANES
_LC = N_LEVELS * 8


def _level_meta():
    meta = []
    for l in range(N_LEVELS):
        res = _RESOLUTIONS[l]
        size = _OFFSETS[l + 1] - _OFFSETS[l]
        dense = (res + 1) ** 3 <= size
        meta.append((res, size, dense, _OFFSETS[l]))
    return meta

_META = _level_meta()


def _sc_body(xt_hbm, valid_hbm, emb_hbm, out_hbm,
             xbuf, vbuf, fracbuf, idxbuf, rbuf, outbuf, sem):
    wid = lax.axis_index("c") * _NS + lax.axis_index("s")
    iota = lax.iota(jnp.int32, _LANES)

    @pl.loop(0, _CHUNKS_PER_W)
    def _chunk(ci):
        pbase = wid * _PTS_PER_W + ci * _CHUNK
        for d in range(3):
            pltpu.sync_copy(xt_hbm.at[pl.ds(d * _N_POINTS + pbase, _CHUNK)],
                            xbuf.at[d])
        pltpu.sync_copy(valid_hbm.at[pl.ds(pbase, _CHUNK)], vbuf)

        for l in range(N_LEVELS):
            res, size, dense, off = _META[l]
            resf = jnp.float32(res)

            @pl.loop(0, _VPC)
            def _idx(v):
                sl = pl.ds(v * _LANES, _LANES)
                x = xbuf[0, sl]
                y = xbuf[1, sl]
                z = xbuf[2, sl]
                px = x * resf
                py = y * resf
                pz = z * resf
                ix = jnp.minimum(px.astype(jnp.int32), res - 1)
                iy = jnp.minimum(py.astype(jnp.int32), res - 1)
                iz = jnp.minimum(pz.astype(jnp.int32), res - 1)
                fracbuf[3 * l + 0, sl] = jnp.minimum(px - ix.astype(jnp.float32), 1.0)
                fracbuf[3 * l + 1, sl] = jnp.minimum(py - iy.astype(jnp.float32), 1.0)
                fracbuf[3 * l + 2, sl] = jnp.minimum(pz - iz.astype(jnp.float32), 1.0)
                if dense:
                    r1 = res + 1
                    base = (ix * r1 + iy) * r1 + iz + off
                    for corner in range(8):
                        cadd = ((corner & 1) * r1 * r1 + ((corner >> 1) & 1) * r1
                                + ((corner >> 2) & 1))
                        idxbuf[8 * l + corner, sl] = base + cadd
                else:
                    mask = jnp.uint32(size - 1)
                    hx0 = ix.astype(jnp.uint32)
                    hx1 = hx0 + jnp.uint32(1)
                    hy0 = iy.astype(jnp.uint32) * jnp.uint32(_PRIME1)
                    hy1 = hy0 + jnp.uint32(_PRIME1)
                    hz0 = iz.astype(jnp.uint32) * jnp.uint32(_PRIME2)
                    hz1 = hz0 + jnp.uint32(_PRIME2)
                    hxs = (hx0, hx1)
                    hys = (hy0, hy1)
                    hzs = (hz0, hz1)
                    for corner in range(8):
                        h = hxs[corner & 1] ^ hys[(corner >> 1) & 1] ^ hzs[(corner >> 2) & 1]
                        idxbuf[8 * l + corner, sl] = (h & mask).astype(jnp.int32) + off

        @pl.loop(0, _LC)
        def _fire(j):
            pltpu.async_copy(emb_hbm.at[idxbuf.at[j]], rbuf.at[j], sem)

        @pl.loop(0, _LC)
        def _drain(j):
            pltpu.make_async_copy(emb_hbm.at[idxbuf.at[j]], rbuf.at[j],
                                  sem).wait()

        for l in range(N_LEVELS):
            @pl.loop(0, _VPC)
            def _interp(v):
                sl = pl.ds(v * _LANES, _LANES)
                fx = fracbuf[3 * l + 0, sl]
                fy = fracbuf[3 * l + 1, sl]
                fz = fracbuf[3 * l + 2, sl]
                vld = vbuf[sl]
                wx = (1.0 - fx, fx)
                wy = (1.0 - fy, fy)
                wz = ((1.0 - fz) * vld, fz * vld)
                f0 = jnp.zeros((_LANES,), jnp.float32)
                f1 = jnp.zeros((_LANES,), jnp.float32)
                for corner in range(8):
                    w = (wx[corner & 1] * wy[(corner >> 1) & 1]
                         * wz[(corner >> 2) & 1])
                    u = lax.bitcast_convert_type(rbuf[8 * l + corner, sl],
                                                 jnp.uint32)
                    e0 = lax.bitcast_convert_type(
                        lax.shift_left(u, jnp.uint32(16)), jnp.float32)
                    e1 = lax.bitcast_convert_type(
                        u & jnp.uint32(0xFFFF0000), jnp.float32)
                    f0 = f0 + w * e0
                    f1 = f1 + w * e1
                pidx = iota + v * _LANES
                plsc.store_scatter(
                    outbuf, [pidx, jnp.full((_LANES,), 2 * l, jnp.int32)], f0)
                plsc.store_scatter(
                    outbuf, [pidx, jnp.full((_LANES,), 2 * l + 1, jnp.int32)], f1)

        pltpu.sync_copy(outbuf, out_hbm.at[pl.ds(pbase, _CHUNK)])


@jax.jit
def kernel(xyz, embeddings, min_xyz, max_xyz):
    scale = 1.0 / (max_xyz - min_xyz)
    xn = (xyz - min_xyz[None, :]) * scale[None, :]
    valid = jnp.all((xn >= 0.0) & (xn <= 1.0), axis=-1).astype(jnp.float32)
    xt = jnp.clip(xn, 0.0, 1.0).T.reshape(-1)
    u0 = lax.bitcast_convert_type(
        embeddings[:, 0].astype(jnp.bfloat16), jnp.uint16).astype(jnp.uint32)
    u1 = lax.bitcast_convert_type(
        embeddings[:, 1].astype(jnp.bfloat16), jnp.uint16).astype(jnp.uint32)
    emb_packed = lax.bitcast_convert_type(u0 | (u1 << 16), jnp.float32)

    mesh = plsc.VectorSubcoreMesh(core_axis_name="c", subcore_axis_name="s",
                                  num_cores=_NC, num_subcores=_NS)
    cp = pltpu.CompilerParams()
    if "needs_layout_passes" in pltpu.CompilerParams.__dataclass_fields__:
        cp = dataclasses.replace(cp, needs_layout_passes=False)
    if "use_tc_tiling_on_sc" in pltpu.CompilerParams.__dataclass_fields__:
        cp = dataclasses.replace(cp, use_tc_tiling_on_sc=False)
    run = pl.kernel(
        _sc_body,
        out_type=jax.ShapeDtypeStruct((_N_POINTS, 2 * N_LEVELS), jnp.float32),
        mesh=mesh,
        scratch_types=[
            pltpu.VMEM((3, _CHUNK), jnp.float32),
            pltpu.VMEM((_CHUNK,), jnp.float32),
            pltpu.VMEM((3 * N_LEVELS, _CHUNK), jnp.float32),
            pltpu.VMEM((_LC, _CHUNK), jnp.int32),
            pltpu.VMEM((_LC, _CHUNK), jnp.float32),
            pltpu.VMEM((_CHUNK, 2 * N_LEVELS), jnp.float32),
            pltpu.SemaphoreType.DMA,
        ],
        compiler_params=cp,
    )
    return run(xt, valid, emb_packed)


# speedup vs baseline: 3.2564x; 1.1664x over previous
import dataclasses
import functools

import jax
import jax.numpy as jnp
from jax import lax
from jax.experimental import pallas as pl
from jax.experimental.pallas import tpu as pltpu
from jax.experimental.pallas import tpu_sc as plsc

N_LEVELS = 16
N_FEAT = 2
_OFFSETS = [0, 4913, 17080, 46871, 126378, 331757, 856045, 1380333, 1904621,
            2428909, 2953197, 3477485, 4001773, 4526061, 5050349, 5574637,
            6098925]
_RESOLUTIONS = [16, 22, 30, 42, 58, 80, 111, 154, 212, 294, 406, 561, 776,
                1072, 1482, 2048]
_PRIME1 = 2654435761
_PRIME2 = 805459861

_N_POINTS = 131072
_NC = 2
_NS = 16
_NW = _NC * _NS
_LANES = 16
_CHUNK = 128
_PTS_PER_W = _N_POINTS // _NW
_CHUNKS_PER_W = _PTS_PER_W // _CHUNK
_VPC = _CHUNK // _LANES
_LC = N_LEVELS * 8
_N_RES = 3
_RES_ROWS = _OFFSETS[_N_RES]
_LC_RES = _N_RES * 8


def _level_meta():
    meta = []
    for l in range(N_LEVELS):
        res = _RESOLUTIONS[l]
        size = _OFFSETS[l + 1] - _OFFSETS[l]
        dense = (res + 1) ** 3 <= size
        meta.append((res, size, dense, _OFFSETS[l]))
    return meta

_META = _level_meta()


def _sc_body(xt_hbm, valid_hbm, emb_hbm, out_hbm,
             xbuf, vbuf, fracbuf, idxbuf, rbuf, outbuf, ltab, sem):
    wid = lax.axis_index("c") * _NS + lax.axis_index("s")
    iota = lax.iota(jnp.int32, _LANES)

    pltpu.sync_copy(emb_hbm.at[pl.ds(0, _RES_ROWS)], ltab)

    @pl.loop(0, _CHUNKS_PER_W)
    def _chunk(ci):
        pbase = wid * _PTS_PER_W + ci * _CHUNK
        for d in range(3):
            pltpu.sync_copy(xt_hbm.at[pl.ds(d * _N_POINTS + pbase, _CHUNK)],
                            xbuf.at[d])
        pltpu.sync_copy(valid_hbm.at[pl.ds(pbase, _CHUNK)], vbuf)

        for l in range(N_LEVELS):
            res, size, dense, off = _META[l]
            resf = jnp.float32(res)

            @pl.loop(0, _VPC)
            def _idx(v):
                sl = pl.ds(v * _LANES, _LANES)
                x = xbuf[0, sl]
                y = xbuf[1, sl]
                z = xbuf[2, sl]
                px = x * resf
                py = y * resf
                pz = z * resf
                ix = jnp.minimum(px.astype(jnp.int32), res - 1)
                iy = jnp.minimum(py.astype(jnp.int32), res - 1)
                iz = jnp.minimum(pz.astype(jnp.int32), res - 1)
                fracbuf[3 * l + 0, sl] = jnp.minimum(px - ix.astype(jnp.float32), 1.0)
                fracbuf[3 * l + 1, sl] = jnp.minimum(py - iy.astype(jnp.float32), 1.0)
                fracbuf[3 * l + 2, sl] = jnp.minimum(pz - iz.astype(jnp.float32), 1.0)
                if dense:
                    r1 = res + 1
                    base = (ix * r1 + iy) * r1 + iz + off
                    for corner in range(8):
                        cadd = ((corner & 1) * r1 * r1 + ((corner >> 1) & 1) * r1
                                + ((corner >> 2) & 1))
                        idxbuf[8 * l + corner, sl] = base + cadd
                else:
                    mask = jnp.uint32(size - 1)
                    hx0 = ix.astype(jnp.uint32)
                    hx1 = hx0 + jnp.uint32(1)
                    hy0 = iy.astype(jnp.uint32) * jnp.uint32(_PRIME1)
                    hy1 = hy0 + jnp.uint32(_PRIME1)
                    hz0 = iz.astype(jnp.uint32) * jnp.uint32(_PRIME2)
                    hz1 = hz0 + jnp.uint32(_PRIME2)
                    hxs = (hx0, hx1)
                    hys = (hy0, hy1)
                    hzs = (hz0, hz1)
                    for corner in range(8):
                        h = hxs[corner & 1] ^ hys[(corner >> 1) & 1] ^ hzs[(corner >> 2) & 1]
                        idxbuf[8 * l + corner, sl] = (h & mask).astype(jnp.int32) + off

        @pl.loop(_LC_RES, _LC)
        def _fire(j):
            pltpu.async_copy(emb_hbm.at[idxbuf.at[j]], rbuf.at[j], sem)

        @pl.loop(_LC_RES, _LC)
        def _drain(j):
            pltpu.make_async_copy(emb_hbm.at[idxbuf.at[j]], rbuf.at[j],
                                  sem).wait()

        for l in range(N_LEVELS):
            @pl.loop(0, _VPC)
            def _interp(v):
                sl = pl.ds(v * _LANES, _LANES)
                fx = fracbuf[3 * l + 0, sl]
                fy = fracbuf[3 * l + 1, sl]
                fz = fracbuf[3 * l + 2, sl]
                vld = vbuf[sl]
                wx = (1.0 - fx, fx)
                wy = (1.0 - fy, fy)
                wz = ((1.0 - fz) * vld, fz * vld)
                f0 = jnp.zeros((_LANES,), jnp.float32)
                f1 = jnp.zeros((_LANES,), jnp.float32)
                for corner in range(8):
                    w = (wx[corner & 1] * wy[(corner >> 1) & 1]
                         * wz[(corner >> 2) & 1])
                    if l < _N_RES:
                        packed = plsc.load_gather(
                            ltab, [idxbuf[8 * l + corner, sl]])
                    else:
                        packed = rbuf[8 * l + corner, sl]
                    u = lax.bitcast_convert_type(packed, jnp.uint32)
                    e0 = lax.bitcast_convert_type(
                        lax.shift_left(u, jnp.uint32(16)), jnp.float32)
                    e1 = lax.bitcast_convert_type(
                        u & jnp.uint32(0xFFFF0000), jnp.float32)
                    f0 = f0 + w * e0
                    f1 = f1 + w * e1
                pidx = iota + v * _LANES
                plsc.store_scatter(
                    outbuf, [pidx, jnp.full((_LANES,), 2 * l, jnp.int32)], f0)
                plsc.store_scatter(
                    outbuf, [pidx, jnp.full((_LANES,), 2 * l + 1, jnp.int32)], f1)

        pltpu.sync_copy(outbuf, out_hbm.at[pl.ds(pbase, _CHUNK)])


@jax.jit
def kernel(xyz, embeddings, min_xyz, max_xyz):
    scale = 1.0 / (max_xyz - min_xyz)
    xn = (xyz - min_xyz[None, :]) * scale[None, :]
    valid = jnp.all((xn >= 0.0) & (xn <= 1.0), axis=-1).astype(jnp.float32)
    xt = jnp.clip(xn, 0.0, 1.0).T.reshape(-1)
    u0 = lax.bitcast_convert_type(
        embeddings[:, 0].astype(jnp.bfloat16), jnp.uint16).astype(jnp.uint32)
    u1 = lax.bitcast_convert_type(
        embeddings[:, 1].astype(jnp.bfloat16), jnp.uint16).astype(jnp.uint32)
    emb_packed = lax.bitcast_convert_type(u0 | (u1 << 16), jnp.float32)

    mesh = plsc.VectorSubcoreMesh(core_axis_name="c", subcore_axis_name="s",
                                  num_cores=_NC, num_subcores=_NS)
    cp = pltpu.CompilerParams()
    if "needs_layout_passes" in pltpu.CompilerParams.__dataclass_fields__:
        cp = dataclasses.replace(cp, needs_layout_passes=False)
    if "use_tc_tiling_on_sc" in pltpu.CompilerParams.__dataclass_fields__:
        cp = dataclasses.replace(cp, use_tc_tiling_on_sc=False)
    run = pl.kernel(
        _sc_body,
        out_type=jax.ShapeDtypeStruct((_N_POINTS, 2 * N_LEVELS), jnp.float32),
        mesh=mesh,
        scratch_types=[
            pltpu.VMEM((3, _CHUNK), jnp.float32),
            pltpu.VMEM((_CHUNK,), jnp.float32),
            pltpu.VMEM((3 * N_LEVELS, _CHUNK), jnp.float32),
            pltpu.VMEM((_LC, _CHUNK), jnp.int32),
            pltpu.VMEM((_LC, _CHUNK), jnp.float32),
            pltpu.VMEM((_CHUNK, 2 * N_LEVELS), jnp.float32),
            pltpu.VMEM((_RES_ROWS,), jnp.float32),
            pltpu.SemaphoreType.DMA,
        ],
        compiler_params=cp,
    )
    return run(xt, valid, emb_packed)


# speedup vs baseline: 3.8237x; 1.1742x over previous
import dataclasses
import functools

import jax
import jax.numpy as jnp
from jax import lax
from jax.experimental import pallas as pl
from jax.experimental.pallas import tpu as pltpu
from jax.experimental.pallas import tpu_sc as plsc

N_LEVELS = 16
N_FEAT = 2
_OFFSETS = [0, 4913, 17080, 46871, 126378, 331757, 856045, 1380333, 1904621,
            2428909, 2953197, 3477485, 4001773, 4526061, 5050349, 5574637,
            6098925]
_RESOLUTIONS = [16, 22, 30, 42, 58, 80, 111, 154, 212, 294, 406, 561, 776,
                1072, 1482, 2048]
_PRIME1 = 2654435761
_PRIME2 = 805459861

_N_POINTS = 131072
_NC = 2
_NS = 16
_NW = _NC * _NS
_LANES = 16
_CHUNK = 128
_PTS_PER_W = _N_POINTS // _NW
_CHUNKS_PER_W = _PTS_PER_W // _CHUNK
_VPC = _CHUNK // _LANES
_LC = N_LEVELS * 8
_N_RES = 3
_RES_ROWS = _OFFSETS[_N_RES]
_LC_RES = _N_RES * 8


def _level_meta():
    meta = []
    for l in range(N_LEVELS):
        res = _RESOLUTIONS[l]
        size = _OFFSETS[l + 1] - _OFFSETS[l]
        dense = (res + 1) ** 3 <= size
        meta.append((res, size, dense, _OFFSETS[l]))
    return meta

_META = _level_meta()


def _sc_body(xt_hbm, valid_hbm, emb_hbm, out_hbm,
             xbuf, vbuf, fracbuf, idxbuf, rbuf, outbuf, ltab, sems):
    wid = lax.axis_index("c") * _NS + lax.axis_index("s")
    iota = lax.iota(jnp.int32, _LANES)

    pltpu.sync_copy(emb_hbm.at[pl.ds(0, _RES_ROWS)], ltab)

    def stage_and_phase1(ci, p):
        pbase = wid * _PTS_PER_W + ci * _CHUNK
        for d in range(3):
            pltpu.sync_copy(xt_hbm.at[pl.ds(d * _N_POINTS + pbase, _CHUNK)],
                            xbuf.at[p, d])
        pltpu.sync_copy(valid_hbm.at[pl.ds(pbase, _CHUNK)], vbuf.at[p])

        for l in range(N_LEVELS):
            res, size, dense, off = _META[l]
            resf = jnp.float32(res)

            @pl.loop(0, _VPC)
            def _idx(v):
                sl = pl.ds(v * _LANES, _LANES)
                x = xbuf[p, 0, sl]
                y = xbuf[p, 1, sl]
                z = xbuf[p, 2, sl]
                px = x * resf
                py = y * resf
                pz = z * resf
                ix = jnp.minimum(px.astype(jnp.int32), res - 1)
                iy = jnp.minimum(py.astype(jnp.int32), res - 1)
                iz = jnp.minimum(pz.astype(jnp.int32), res - 1)
                fracbuf[p, 3 * l + 0, sl] = jnp.minimum(px - ix.astype(jnp.float32), 1.0)
                fracbuf[p, 3 * l + 1, sl] = jnp.minimum(py - iy.astype(jnp.float32), 1.0)
                fracbuf[p, 3 * l + 2, sl] = jnp.minimum(pz - iz.astype(jnp.float32), 1.0)
                if dense:
                    r1 = res + 1
                    base = (ix * r1 + iy) * r1 + iz + off
                    for corner in range(8):
                        cadd = ((corner & 1) * r1 * r1 + ((corner >> 1) & 1) * r1
                                + ((corner >> 2) & 1))
                        idxbuf[p, 8 * l + corner, sl] = base + cadd
                else:
                    mask = jnp.uint32(size - 1)
                    hx0 = ix.astype(jnp.uint32)
                    hx1 = hx0 + jnp.uint32(1)
                    hy0 = iy.astype(jnp.uint32) * jnp.uint32(_PRIME1)
                    hy1 = hy0 + jnp.uint32(_PRIME1)
                    hz0 = iz.astype(jnp.uint32) * jnp.uint32(_PRIME2)
                    hz1 = hz0 + jnp.uint32(_PRIME2)
                    hxs = (hx0, hx1)
                    hys = (hy0, hy1)
                    hzs = (hz0, hz1)
                    for corner in range(8):
                        h = hxs[corner & 1] ^ hys[(corner >> 1) & 1] ^ hzs[(corner >> 2) & 1]
                        idxbuf[p, 8 * l + corner, sl] = (h & mask).astype(jnp.int32) + off

    def fire(p):
        @pl.loop(_LC_RES, _LC)
        def _fire(j):
            pltpu.async_copy(emb_hbm.at[idxbuf.at[p, j]],
                             rbuf.at[p, j - _LC_RES], sems.at[p])

    def drain(p):
        @pl.loop(_LC_RES, _LC)
        def _drain(j):
            pltpu.make_async_copy(emb_hbm.at[idxbuf.at[p, j]],
                                  rbuf.at[p, j - _LC_RES], sems.at[p]).wait()

    def interp_out(ci, p):
        pbase = wid * _PTS_PER_W + ci * _CHUNK
        for l in range(N_LEVELS):
            @pl.loop(0, _VPC)
            def _interp(v):
                sl = pl.ds(v * _LANES, _LANES)
                fx = fracbuf[p, 3 * l + 0, sl]
                fy = fracbuf[p, 3 * l + 1, sl]
                fz = fracbuf[p, 3 * l + 2, sl]
                vld = vbuf[p, sl]
                wx = (1.0 - fx, fx)
                wy = (1.0 - fy, fy)
                wz = ((1.0 - fz) * vld, fz * vld)
                f0 = jnp.zeros((_LANES,), jnp.float32)
                f1 = jnp.zeros((_LANES,), jnp.float32)
                for corner in range(8):
                    w = (wx[corner & 1] * wy[(corner >> 1) & 1]
                         * wz[(corner >> 2) & 1])
                    if l < _N_RES:
                        packed = plsc.load_gather(
                            ltab, [idxbuf[p, 8 * l + corner, sl]])
                    else:
                        packed = rbuf[p, 8 * l + corner - _LC_RES, sl]
                    u = lax.bitcast_convert_type(packed, jnp.uint32)
                    e0 = lax.bitcast_convert_type(
                        lax.shift_left(u, jnp.uint32(16)), jnp.float32)
                    e1 = lax.bitcast_convert_type(
                        u & jnp.uint32(0xFFFF0000), jnp.float32)
                    f0 = f0 + w * e0
                    f1 = f1 + w * e1
                pidx = iota + v * _LANES
                plsc.store_scatter(
                    outbuf, [pidx, jnp.full((_LANES,), 2 * l, jnp.int32)], f0)
                plsc.store_scatter(
                    outbuf, [pidx, jnp.full((_LANES,), 2 * l + 1, jnp.int32)], f1)

        pltpu.sync_copy(outbuf, out_hbm.at[pl.ds(pbase, _CHUNK)])

    @pl.loop(0, _CHUNKS_PER_W + 1)
    def _main(ci):
        @pl.when(ci < _CHUNKS_PER_W)
        def _():
            stage_and_phase1(ci, ci & 1)
            fire(ci & 1)

        @pl.when(ci > 0)
        def _():
            drain((ci - 1) & 1)
            interp_out(ci - 1, (ci - 1) & 1)


@jax.jit
def kernel(xyz, embeddings, min_xyz, max_xyz):
    scale = 1.0 / (max_xyz - min_xyz)
    xn = (xyz - min_xyz[None, :]) * scale[None, :]
    valid = jnp.all((xn >= 0.0) & (xn <= 1.0), axis=-1).astype(jnp.float32)
    xt = jnp.clip(xn, 0.0, 1.0).T.reshape(-1)
    u0 = lax.bitcast_convert_type(
        embeddings[:, 0].astype(jnp.bfloat16), jnp.uint16).astype(jnp.uint32)
    u1 = lax.bitcast_convert_type(
        embeddings[:, 1].astype(jnp.bfloat16), jnp.uint16).astype(jnp.uint32)
    emb_packed = lax.bitcast_convert_type(u0 | (u1 << 16), jnp.float32)

    mesh = plsc.VectorSubcoreMesh(core_axis_name="c", subcore_axis_name="s",
                                  num_cores=_NC, num_subcores=_NS)
    cp = pltpu.CompilerParams()
    if "needs_layout_passes" in pltpu.CompilerParams.__dataclass_fields__:
        cp = dataclasses.replace(cp, needs_layout_passes=False)
    if "use_tc_tiling_on_sc" in pltpu.CompilerParams.__dataclass_fields__:
        cp = dataclasses.replace(cp, use_tc_tiling_on_sc=False)
    run = pl.kernel(
        _sc_body,
        out_type=jax.ShapeDtypeStruct((_N_POINTS, 2 * N_LEVELS), jnp.float32),
        mesh=mesh,
        scratch_types=[
            pltpu.VMEM((2, 3, _CHUNK), jnp.float32),
            pltpu.VMEM((2, _CHUNK), jnp.float32),
            pltpu.VMEM((2, 3 * N_LEVELS, _CHUNK), jnp.float32),
            pltpu.VMEM((2, _LC, _CHUNK), jnp.int32),
            pltpu.VMEM((2, _LC - _LC_RES, _CHUNK), jnp.float32),
            pltpu.VMEM((_CHUNK, 2 * N_LEVELS), jnp.float32),
            pltpu.VMEM((_RES_ROWS,), jnp.float32),
            pltpu.SemaphoreType.DMA((2,)),
        ],
        compiler_params=cp,
    )
    return run(xt, valid, emb_packed)


# speedup vs baseline: 4.4010x; 1.1510x over previous
import dataclasses
import functools

import jax
import jax.numpy as jnp
from jax import lax
from jax.experimental import pallas as pl
from jax.experimental.pallas import tpu as pltpu
from jax.experimental.pallas import tpu_sc as plsc

N_LEVELS = 16
N_FEAT = 2
_OFFSETS = [0, 4913, 17080, 46871, 126378, 331757, 856045, 1380333, 1904621,
            2428909, 2953197, 3477485, 4001773, 4526061, 5050349, 5574637,
            6098925]
_RESOLUTIONS = [16, 22, 30, 42, 58, 80, 111, 154, 212, 294, 406, 561, 776,
                1072, 1482, 2048]
_PRIME1 = 2654435761
_PRIME2 = 805459861

_N_POINTS = 131072
_NC = 2
_NS = 16
_NW = _NC * _NS
_LANES = 16
_CHUNK = 128
_PTS_PER_W = _N_POINTS // _NW
_CHUNKS_PER_W = _PTS_PER_W // _CHUNK
_VPC = _CHUNK // _LANES
_LC = N_LEVELS * 8
_N_RES = 3
_RES_ROWS = _OFFSETS[_N_RES]
_LC_RES = _N_RES * 8


def _level_meta():
    meta = []
    for l in range(N_LEVELS):
        res = _RESOLUTIONS[l]
        size = _OFFSETS[l + 1] - _OFFSETS[l]
        dense = (res + 1) ** 3 <= size
        meta.append((res, size, dense, _OFFSETS[l]))
    return meta

_META = _level_meta()


def _sc_body(xt_hbm, valid_hbm, emb_hbm, out_hbm,
             xbuf, vbuf, fracbuf, idxbuf, rbuf, outbuf, ltab, sems):
    wid = lax.axis_index("c") * _NS + lax.axis_index("s")
    iota = lax.iota(jnp.int32, _LANES)

    pltpu.sync_copy(emb_hbm.at[pl.ds(0, _RES_ROWS)], ltab)

    def stage_and_phase1(ci, p):
        pbase = wid * _PTS_PER_W + ci * _CHUNK
        for d in range(3):
            pltpu.sync_copy(xt_hbm.at[pl.ds(d * _N_POINTS + pbase, _CHUNK)],
                            xbuf.at[p, d])
        pltpu.sync_copy(valid_hbm.at[pl.ds(pbase, _CHUNK)], vbuf.at[p])

        for l in range(N_LEVELS):
            res, size, dense, off = _META[l]
            resf = jnp.float32(res)

            @pl.loop(0, _VPC)
            def _idx(v):
                sl = pl.ds(v * _LANES, _LANES)
                x = xbuf[p, 0, sl]
                y = xbuf[p, 1, sl]
                z = xbuf[p, 2, sl]
                px = x * resf
                py = y * resf
                pz = z * resf
                ix = jnp.minimum(px.astype(jnp.int32), res - 1)
                iy = jnp.minimum(py.astype(jnp.int32), res - 1)
                iz = jnp.minimum(pz.astype(jnp.int32), res - 1)
                fracbuf[p, 3 * l + 0, sl] = jnp.minimum(px - ix.astype(jnp.float32), 1.0)
                fracbuf[p, 3 * l + 1, sl] = jnp.minimum(py - iy.astype(jnp.float32), 1.0)
                fracbuf[p, 3 * l + 2, sl] = jnp.minimum(pz - iz.astype(jnp.float32), 1.0)
                if dense:
                    r1 = res + 1
                    base = (ix * r1 + iy) * r1 + iz + off
                    for corner in range(8):
                        cadd = ((corner & 1) * r1 * r1 + ((corner >> 1) & 1) * r1
                                + ((corner >> 2) & 1))
                        idxbuf[p, pl.ds((8 * l + corner) * _CHUNK + v * _LANES,
                                        _LANES)] = base + cadd
                else:
                    mask = jnp.uint32(size - 1)
                    hx0 = ix.astype(jnp.uint32)
                    hx1 = hx0 + jnp.uint32(1)
                    hy0 = iy.astype(jnp.uint32) * jnp.uint32(_PRIME1)
                    hy1 = hy0 + jnp.uint32(_PRIME1)
                    hz0 = iz.astype(jnp.uint32) * jnp.uint32(_PRIME2)
                    hz1 = hz0 + jnp.uint32(_PRIME2)
                    hxs = (hx0, hx1)
                    hys = (hy0, hy1)
                    hzs = (hz0, hz1)
                    for corner in range(8):
                        h = hxs[corner & 1] ^ hys[(corner >> 1) & 1] ^ hzs[(corner >> 2) & 1]
                        idxbuf[p, pl.ds((8 * l + corner) * _CHUNK + v * _LANES,
                                        _LANES)] = (h & mask).astype(jnp.int32) + off

    _HBM_IDX = pl.ds(_LC_RES * _CHUNK, (_LC - _LC_RES) * _CHUNK)

    def fire(p):
        pltpu.async_copy(emb_hbm.at[idxbuf.at[p, _HBM_IDX]], rbuf.at[p],
                         sems.at[p])

    def drain(p):
        pltpu.make_async_copy(emb_hbm.at[idxbuf.at[p, _HBM_IDX]], rbuf.at[p],
                              sems.at[p]).wait()

    def interp_out(ci, p):
        pbase = wid * _PTS_PER_W + ci * _CHUNK
        for l in range(N_LEVELS):
            @pl.loop(0, _VPC)
            def _interp(v):
                sl = pl.ds(v * _LANES, _LANES)
                fx = fracbuf[p, 3 * l + 0, sl]
                fy = fracbuf[p, 3 * l + 1, sl]
                fz = fracbuf[p, 3 * l + 2, sl]
                vld = vbuf[p, sl]
                wx = (1.0 - fx, fx)
                wy = (1.0 - fy, fy)
                wz = ((1.0 - fz) * vld, fz * vld)
                f0 = jnp.zeros((_LANES,), jnp.float32)
                f1 = jnp.zeros((_LANES,), jnp.float32)
                for corner in range(8):
                    w = (wx[corner & 1] * wy[(corner >> 1) & 1]
                         * wz[(corner >> 2) & 1])
                    if l < _N_RES:
                        packed = plsc.load_gather(
                            ltab,
                            [idxbuf[p, pl.ds((8 * l + corner) * _CHUNK
                                             + v * _LANES, _LANES)]])
                    else:
                        packed = rbuf[p, pl.ds((8 * l + corner - _LC_RES)
                                               * _CHUNK + v * _LANES, _LANES)]
                    u = lax.bitcast_convert_type(packed, jnp.uint32)
                    e0 = lax.bitcast_convert_type(
                        lax.shift_left(u, jnp.uint32(16)), jnp.float32)
                    e1 = lax.bitcast_convert_type(
                        u & jnp.uint32(0xFFFF0000), jnp.float32)
                    f0 = f0 + w * e0
                    f1 = f1 + w * e1
                pidx = iota + v * _LANES
                plsc.store_scatter(
                    outbuf, [pidx, jnp.full((_LANES,), 2 * l, jnp.int32)], f0)
                plsc.store_scatter(
                    outbuf, [pidx, jnp.full((_LANES,), 2 * l + 1, jnp.int32)], f1)

        pltpu.sync_copy(outbuf, out_hbm.at[pl.ds(pbase, _CHUNK)])

    @pl.loop(0, _CHUNKS_PER_W + 1)
    def _main(ci):
        @pl.when(ci < _CHUNKS_PER_W)
        def _():
            stage_and_phase1(ci, ci & 1)
            fire(ci & 1)

        @pl.when(ci > 0)
        def _():
            drain((ci - 1) & 1)
            interp_out(ci - 1, (ci - 1) & 1)


@jax.jit
def kernel(xyz, embeddings, min_xyz, max_xyz):
    scale = 1.0 / (max_xyz - min_xyz)
    xn = (xyz - min_xyz[None, :]) * scale[None, :]
    valid = jnp.all((xn >= 0.0) & (xn <= 1.0), axis=-1).astype(jnp.float32)
    xt = jnp.clip(xn, 0.0, 1.0).T.reshape(-1)
    u0 = lax.bitcast_convert_type(
        embeddings[:, 0].astype(jnp.bfloat16), jnp.uint16).astype(jnp.uint32)
    u1 = lax.bitcast_convert_type(
        embeddings[:, 1].astype(jnp.bfloat16), jnp.uint16).astype(jnp.uint32)
    emb_packed = lax.bitcast_convert_type(u0 | (u1 << 16), jnp.float32)

    mesh = plsc.VectorSubcoreMesh(core_axis_name="c", subcore_axis_name="s",
                                  num_cores=_NC, num_subcores=_NS)
    cp = pltpu.CompilerParams()
    if "needs_layout_passes" in pltpu.CompilerParams.__dataclass_fields__:
        cp = dataclasses.replace(cp, needs_layout_passes=False)
    if "use_tc_tiling_on_sc" in pltpu.CompilerParams.__dataclass_fields__:
        cp = dataclasses.replace(cp, use_tc_tiling_on_sc=False)
    run = pl.kernel(
        _sc_body,
        out_type=jax.ShapeDtypeStruct((_N_POINTS, 2 * N_LEVELS), jnp.float32),
        mesh=mesh,
        scratch_types=[
            pltpu.VMEM((2, 3, _CHUNK), jnp.float32),
            pltpu.VMEM((2, _CHUNK), jnp.float32),
            pltpu.VMEM((2, 3 * N_LEVELS, _CHUNK), jnp.float32),
            pltpu.VMEM((2, _LC * _CHUNK), jnp.int32),
            pltpu.VMEM((2, (_LC - _LC_RES) * _CHUNK), jnp.float32),
            pltpu.VMEM((_CHUNK, 2 * N_LEVELS), jnp.float32),
            pltpu.VMEM((_RES_ROWS,), jnp.float32),
            pltpu.SemaphoreType.DMA((2,)),
        ],
        compiler_params=cp,
    )
    return run(xt, valid, emb_packed)
